# Initial kernel scaffold; baseline (speedup 1.0000x reference)
import jax
import jax.numpy as jnp
from jax.experimental import pallas as pl


def kernel(inputs):
    raise NotImplementedError("write your pallas kernel here")


# speedup vs baseline: 4.0785x; 4.0785x over previous
import functools

import jax
import jax.numpy as jnp
from jax import lax
from jax.experimental import pallas as pl
from jax.experimental.pallas import tpu as pltpu
from jax.experimental.pallas import tpu_sc as plsc

B = 64
N = 8192
K = 256
L = 16
NV = N // L
NC = 2
NW = 32
ROWS_PER_W = B // NW
SHIFTS = (24, 16, 8, 0)


def _level_select(tot16, kk, lane, zeros_i):
    cs = plsc.cumsum(lax.rev(tot16, (0,)))
    m = cs >= kk
    npc = plsc.all_reduce_population_count(m)
    slot_v = npc - 1
    cgt = jnp.sum(jnp.where(lane > slot_v, tot16, zeros_i))
    slot_s = jnp.max(slot_v) if getattr(slot_v, "ndim", 0) else slot_v
    return slot_s, cgt


def _tec_body(x_hbm, out_hbm, rows_v, key_v, hist_v):
    wid = lax.axis_index("s") * NC + lax.axis_index("c")
    base = wid * ROWS_PER_W
    pltpu.sync_copy(x_hbm.at[pl.ds(base, ROWS_PER_W)], rows_v)

    lane = lax.iota(jnp.int32, L)
    ones_i = jnp.full((L,), 1, jnp.int32)
    zeros_i = jnp.zeros((L,), jnp.int32)
    ones_f = jnp.full((L,), 1.0, jnp.float32)
    zeros_f = jnp.zeros((L,), jnp.float32)

    for r in range(ROWS_PER_W):
        def kbody(i, _):
            x = rows_v[r, pl.ds(i * L, L)]
            b = lax.bitcast_convert_type(x, jnp.int32)
            m = lax.shift_right_arithmetic(b, 31) & jnp.int32(0x7FFFFFFF)
            key_v[pl.ds(i * L, L)] = b ^ m
            return 0

        lax.fori_loop(0, NV, kbody, 0, unroll=8)

        prefix = jnp.int32(0)
        kk = jnp.int32(K)
        for s in SHIFTS:
            if s + 8 < 32:
                hi = (~((1 << (s + 8)) - 1)) & 0xFFFFFFFF
                himask = jnp.int32(hi - (1 << 32) if hi >= (1 << 31) else hi)
            else:
                himask = jnp.int32(0)

            def zbody(i, _):
                hist_v[pl.ds(i * L, L)] = zeros_i
                return 0

            lax.fori_loop(0, 256, zbody, 0, unroll=8)

            def sbody(i, _):
                key = key_v[pl.ds(i * L, L)]
                msk = (key & himask) == prefix
                digit = lax.shift_right_arithmetic(key, s) & jnp.int32(0xFF)
                if s == 24:
                    digit = digit ^ jnp.int32(0x80)
                idx = digit * L + lane
                plsc.addupdate_scatter(hist_v, [idx], ones_i, mask=msk)
                return 0

            lax.fori_loop(0, NV, sbody, 0, unroll=4)

            tot_chunk = zeros_i
            for c in range(16):
                acc = zeros_i
                for j in range(16):
                    acc = acc + hist_v[pl.ds((c * 16 + j) * L, L)]
                tc = jnp.sum(acc)
                tot_chunk = jnp.where(lane == c, tc, tot_chunk)
            cslot, cgt_c = _level_select(tot_chunk, kk, lane, zeros_i)

            ibase = cslot * 256
            tot_in = zeros_i
            for j in range(16):
                tj = jnp.sum(hist_v[pl.ds(ibase + j * L, L)])
                tot_in = jnp.where(lane == j, tj, tot_in)
            kk2 = kk - cgt_c
            islot, cgt_i = _level_select(tot_in, kk2, lane, zeros_i)

            d = cslot * 16 + islot
            if s == 24:
                d = d ^ jnp.int32(0x80)
            prefix = prefix | lax.shift_left(d, s)
            kk = kk2 - cgt_i

        def mbody(i, _):
            key = key_v[pl.ds(i * L, L)]
            rows_v[r, pl.ds(i * L, L)] = jnp.where(key >= prefix, ones_f, zeros_f)
            return 0

        lax.fori_loop(0, NV, mbody, 0, unroll=8)

    pltpu.sync_copy(rows_v, out_hbm.at[pl.ds(base, ROWS_PER_W)])


_mesh = plsc.VectorSubcoreMesh(core_axis_name="c", subcore_axis_name="s")


@functools.partial(
    pl.kernel,
    out_type=jax.ShapeDtypeStruct((B, N), jnp.float32),
    mesh=_mesh,
    scratch_types=[
        pltpu.VMEM((ROWS_PER_W, N), jnp.float32),
        pltpu.VMEM((N,), jnp.int32),
        pltpu.VMEM((256 * L,), jnp.int32),
    ],
    compiler_params=pltpu.CompilerParams(needs_layout_passes=False),
)
def _topk_mask(x_hbm, out_hbm, rows_v, key_v, hist_v):
    _tec_body(x_hbm, out_hbm, rows_v, key_v, hist_v)


def kernel(inputs):
    return _topk_mask(inputs)


# speedup vs baseline: 4.2749x; 1.0482x over previous
import functools

import jax
import jax.numpy as jnp
from jax import lax
from jax.experimental import pallas as pl
from jax.experimental.pallas import tpu as pltpu
from jax.experimental.pallas import tpu_sc as plsc

B = 64
N = 8192
K = 256
L = 16
NV = N // L
NC = 2
NW = 32
ROWS_PER_W = B // NW
SHIFTS = (24, 16, 8, 0)


def _level_select(tot16, kk, lane, zeros_i):
    cs = plsc.cumsum(lax.rev(tot16, (0,)))
    m = cs >= kk
    npc = plsc.all_reduce_population_count(m)
    slot_v = npc - 1
    cgt = jnp.sum(jnp.where(lane > slot_v, tot16, zeros_i))
    slot_s = jnp.max(slot_v) if getattr(slot_v, "ndim", 0) else slot_v
    return slot_s, cgt


def _tec_body(x_hbm, out_hbm, rows_v, key_v, hist_v):
    wid = lax.axis_index("s") * NC + lax.axis_index("c")
    base = wid * ROWS_PER_W
    pltpu.sync_copy(x_hbm.at[pl.ds(base, ROWS_PER_W)], rows_v)

    lane = lax.iota(jnp.int32, L)
    ones_i = jnp.full((L,), 1, jnp.int32)
    zeros_i = jnp.zeros((L,), jnp.int32)
    ones_f = jnp.full((L,), 1.0, jnp.float32)
    zeros_f = jnp.zeros((L,), jnp.float32)

    for r in range(ROWS_PER_W):
        prefix = jnp.int32(0)
        kk = jnp.int32(K)
        for s in SHIFTS:
            if s + 8 < 32:
                hi = (~((1 << (s + 8)) - 1)) & 0xFFFFFFFF
                himask = jnp.int32(hi - (1 << 32) if hi >= (1 << 31) else hi)
            else:
                himask = jnp.int32(0)

            def zbody(i, _):
                hist_v[pl.ds(i * L, L)] = zeros_i
                return 0

            lax.fori_loop(0, 256, zbody, 0, unroll=16)

            if s == 24:

                def sbody(i, _):
                    x = rows_v[r, pl.ds(i * L, L)]
                    b = lax.bitcast_convert_type(x, jnp.int32)
                    m = lax.shift_right_arithmetic(b, 31) & jnp.int32(0x7FFFFFFF)
                    key = b ^ m
                    key_v[pl.ds(i * L, L)] = key
                    digit = lax.shift_right_arithmetic(key, 24) & jnp.int32(0xFF)
                    digit = digit ^ jnp.int32(0x80)
                    idx = digit * L + lane
                    plsc.addupdate_scatter(hist_v, [idx], ones_i)
                    return 0

            else:

                def sbody(i, _):
                    key = key_v[pl.ds(i * L, L)]
                    msk = (key & himask) == prefix
                    digit = lax.shift_right_arithmetic(key, s) & jnp.int32(0xFF)
                    idx = digit * L + lane
                    plsc.addupdate_scatter(hist_v, [idx], ones_i, mask=msk)
                    return 0

            lax.fori_loop(0, NV, sbody, 0, unroll=8)

            tot_chunk = zeros_i
            for c in range(16):
                acc = zeros_i
                for j in range(16):
                    acc = acc + hist_v[pl.ds((c * 16 + j) * L, L)]
                tc = jnp.sum(acc)
                tot_chunk = jnp.where(lane == c, tc, tot_chunk)
            cslot, cgt_c = _level_select(tot_chunk, kk, lane, zeros_i)

            ibase = cslot * 256
            tot_in = zeros_i
            for j in range(16):
                tj = jnp.sum(hist_v[pl.ds(ibase + j * L, L)])
                tot_in = jnp.where(lane == j, tj, tot_in)
            kk2 = kk - cgt_c
            islot, cgt_i = _level_select(tot_in, kk2, lane, zeros_i)

            d = cslot * 16 + islot
            if s == 24:
                d = d ^ jnp.int32(0x80)
            prefix = prefix | lax.shift_left(d, s)
            kk = kk2 - cgt_i

        def mbody(i, _):
            key = key_v[pl.ds(i * L, L)]
            rows_v[r, pl.ds(i * L, L)] = jnp.where(key >= prefix, ones_f, zeros_f)
            return 0

        lax.fori_loop(0, NV, mbody, 0, unroll=8)

    pltpu.sync_copy(rows_v, out_hbm.at[pl.ds(base, ROWS_PER_W)])


_mesh = plsc.VectorSubcoreMesh(core_axis_name="c", subcore_axis_name="s")


@functools.partial(
    pl.kernel,
    out_type=jax.ShapeDtypeStruct((B, N), jnp.float32),
    mesh=_mesh,
    scratch_types=[
        pltpu.VMEM((ROWS_PER_W, N), jnp.float32),
        pltpu.VMEM((N,), jnp.int32),
        pltpu.VMEM((256 * L,), jnp.int32),
    ],
    compiler_params=pltpu.CompilerParams(needs_layout_passes=False),
)
def _topk_mask(x_hbm, out_hbm, rows_v, key_v, hist_v):
    _tec_body(x_hbm, out_hbm, rows_v, key_v, hist_v)


def kernel(inputs):
    return _topk_mask(inputs)


# speedup vs baseline: 7.5149x; 1.7579x over previous
import functools

import jax
import jax.numpy as jnp
from jax import lax
from jax.experimental import pallas as pl
from jax.experimental.pallas import tpu as pltpu
from jax.experimental.pallas import tpu_sc as plsc

B = 64
N = 8192
K = 256
L = 16
NV = N // L
NC = 2
NW = 32
ROWS_PER_W = B // NW
SHIFTS = (24, 16, 8, 0)


def _tec_body(x_hbm, out_hbm, rows_v, key_v, hist_v, tr_v):
    wid = lax.axis_index("s") * NC + lax.axis_index("c")
    base = wid * ROWS_PER_W
    pltpu.sync_copy(x_hbm.at[pl.ds(base, ROWS_PER_W)], rows_v)

    lane = lax.iota(jnp.int32, L)
    ones_i = jnp.full((L,), 1, jnp.int32)
    zeros_i = jnp.zeros((L,), jnp.int32)
    ones_f = jnp.full((L,), 1.0, jnp.float32)
    zeros_f = jnp.zeros((L,), jnp.float32)

    def level_select(tot16, kk):
        cs = plsc.cumsum(lax.rev(tot16, (0,)))
        m = cs >= kk
        npc = plsc.all_reduce_population_count(m)
        slot_v = npc - 1
        i0 = 16 - npc
        csm1 = cs.at[jnp.maximum(i0 - 1, 0)].get(mode="promise_in_bounds")
        cgt = jnp.where(i0 == 0, zeros_i, csm1)
        return slot_v, cgt

    def totals(offset_fn, vecs_per_group):
        accs = []
        for g in range(16):
            acc = hist_v[pl.ds(offset_fn(g, 0), L)]
            for j in range(1, vecs_per_group):
                acc = acc + hist_v[pl.ds(offset_fn(g, j), L)]
            accs.append(acc)
        for g in range(16):
            plsc.store_scatter(tr_v, [lane * 16 + g], accs[g])
        tot = tr_v[pl.ds(0, L)]
        for l in range(1, 16):
            tot = tot + tr_v[pl.ds(l * L, L)]
        return tot

    for r in range(ROWS_PER_W):
        prefix = zeros_i
        kk = jnp.full((L,), K, jnp.int32)
        for s in SHIFTS:
            if s + 8 < 32:
                hi = (~((1 << (s + 8)) - 1)) & 0xFFFFFFFF
                himask = jnp.int32(hi - (1 << 32) if hi >= (1 << 31) else hi)
            else:
                himask = jnp.int32(0)

            @plsc.parallel_loop(0, 256, unroll=8)
            def _(i):
                hist_v[pl.ds(i * L, L)] = zeros_i

            if s == 24:

                @plsc.parallel_loop(0, NV, unroll=8)
                def _(i):
                    x = rows_v[r, pl.ds(i * L, L)]
                    b = lax.bitcast_convert_type(x, jnp.int32)
                    m = lax.shift_right_arithmetic(b, 31) & jnp.int32(0x7FFFFFFF)
                    key = b ^ m
                    key_v[pl.ds(i * L, L)] = key
                    digit = lax.shift_right_arithmetic(key, 24) & jnp.int32(0xFF)
                    digit = digit ^ jnp.int32(0x80)
                    plsc.addupdate_scatter(hist_v, [digit * L + lane], ones_i)

            else:

                @plsc.parallel_loop(0, NV, unroll=8)
                def _(i):
                    key = key_v[pl.ds(i * L, L)]
                    msk = (key & himask) == prefix
                    digit = lax.shift_right_arithmetic(key, s) & jnp.int32(0xFF)
                    plsc.addupdate_scatter(
                        hist_v, [digit * L + lane], ones_i, mask=msk)

            tot_chunk = totals(lambda g, j: (g * 16 + j) * L, 16)
            cslot_v, cgt_c = level_select(tot_chunk, kk)
            cslot_s = jnp.max(cslot_v)

            ibase = cslot_s * 256
            tot_in = totals(lambda g, j: ibase + g * L, 1)
            kk2 = kk - cgt_c
            islot_v, cgt_i = level_select(tot_in, kk2)

            d = cslot_v * 16 + islot_v
            if s == 24:
                d = d ^ jnp.int32(0x80)
            prefix = prefix | jnp.left_shift(d, s)
            kk = kk2 - cgt_i

        @plsc.parallel_loop(0, NV, unroll=8)
        def _(i):
            key = key_v[pl.ds(i * L, L)]
            rows_v[r, pl.ds(i * L, L)] = jnp.where(key >= prefix, ones_f, zeros_f)

    pltpu.sync_copy(rows_v, out_hbm.at[pl.ds(base, ROWS_PER_W)])


_mesh = plsc.VectorSubcoreMesh(core_axis_name="c", subcore_axis_name="s")


@functools.partial(
    pl.kernel,
    out_type=jax.ShapeDtypeStruct((B, N), jnp.float32),
    mesh=_mesh,
    scratch_types=[
        pltpu.VMEM((ROWS_PER_W, N), jnp.float32),
        pltpu.VMEM((N,), jnp.int32),
        pltpu.VMEM((256 * L,), jnp.int32),
        pltpu.VMEM((256,), jnp.int32),
    ],
    compiler_params=pltpu.CompilerParams(needs_layout_passes=False),
)
def _topk_mask(x_hbm, out_hbm, rows_v, key_v, hist_v, tr_v):
    _tec_body(x_hbm, out_hbm, rows_v, key_v, hist_v, tr_v)


def kernel(inputs):
    return _topk_mask(inputs)


# speedup vs baseline: 8.8709x; 1.1804x over previous
import functools

import jax
import jax.numpy as jnp
from jax import lax
from jax.experimental import pallas as pl
from jax.experimental.pallas import tpu as pltpu
from jax.experimental.pallas import tpu_sc as plsc

B = 64
N = 8192
K = 256
L = 16
NV = N // L
NC = 2
NW = 32
ROWS_PER_W = B // NW


def _tec_body(x_hbm, out_hbm, row_v, key_v, hist_v, tr_v):
    wid = lax.axis_index("s") * NC + lax.axis_index("c")
    base = wid * ROWS_PER_W

    lane = lax.iota(jnp.int32, L)
    ones_i = jnp.full((L,), 1, jnp.int32)
    zeros_i = jnp.zeros((L,), jnp.int32)
    ones_f = jnp.full((L,), 1.0, jnp.float32)
    zeros_f = jnp.zeros((L,), jnp.float32)

    def level_select(tot16, kk):
        cs = plsc.cumsum(lax.rev(tot16, (0,)))
        m = cs >= kk
        npc = plsc.all_reduce_population_count(m)
        slot_v = npc - 1
        i0 = 16 - npc
        csm1 = cs.at[jnp.maximum(i0 - 1, 0)].get(mode="promise_in_bounds")
        cgt = jnp.where(i0 == 0, zeros_i, csm1)
        return slot_v, cgt

    def totals(offset_fn, vecs_per_group):
        accs = []
        for g in range(16):
            acc = hist_v[pl.ds(offset_fn(g, 0), L)]
            for j in range(1, vecs_per_group):
                acc = acc + hist_v[pl.ds(offset_fn(g, j), L)]
            accs.append(acc)
        for g in range(16):
            plsc.store_scatter(tr_v, [lane * 16 + g], accs[g])
        tot = tr_v[pl.ds(0, L)]
        for l in range(1, 16):
            tot = tot + tr_v[pl.ds(l * L, L)]
        return tot

    def select_digit(kk):
        tot_chunk = totals(lambda g, j: (g * 16 + j) * L, 16)
        cslot_v, cgt_c = level_select(tot_chunk, kk)
        cslot_s = jnp.max(cslot_v)
        ibase = cslot_s * 256
        tot_in = totals(lambda g, j: ibase + g * L, 1)
        kk2 = kk - cgt_c
        islot_v, cgt_i = level_select(tot_in, kk2)
        return cslot_v * 16 + islot_v, cgt_c + cgt_i

    def zero_hist():
        @plsc.parallel_loop(0, 256, unroll=8)
        def _(i):
            hist_v[pl.ds(i * L, L)] = zeros_i

    def row_body(r, _):
        pltpu.sync_copy(x_hbm.at[pl.ds(base + r, 1)], row_v)

        zero_hist()

        @plsc.parallel_loop(0, NV, unroll=8)
        def _(i):
            x = row_v[0, pl.ds(i * L, L)]
            b = lax.bitcast_convert_type(x, jnp.int32)
            m = lax.shift_right_arithmetic(b, 31) & jnp.int32(0x7FFFFFFF)
            key = b ^ m
            key_v[pl.ds(i * L, L)] = key
            digit = lax.shift_right_arithmetic(key, 24) & jnp.int32(0xFF)
            digit = digit ^ jnp.int32(0x80)
            plsc.addupdate_scatter(hist_v, [digit * L + lane], ones_i)

        kk = jnp.full((L,), K, jnp.int32)
        d0, cgt0 = select_digit(kk)
        prefix = jnp.left_shift(d0 ^ jnp.int32(0x80), 24)
        kk = kk - cgt0

        def pass_body(p, carry):
            prefix, kk = carry
            shift = 24 - 8 * p
            sh8 = shift + 8
            shift_v = jnp.full((L,), 1, jnp.int32) * shift
            sh8_v = jnp.full((L,), 1, jnp.int32) * sh8
            pref_hi = lax.shift_right_arithmetic(prefix, sh8_v)
            zero_hist()

            @plsc.parallel_loop(0, NV, unroll=8)
            def _(i):
                key = key_v[pl.ds(i * L, L)]
                msk = lax.shift_right_arithmetic(key, sh8_v) == pref_hi
                digit = lax.shift_right_arithmetic(key, shift_v) & jnp.int32(0xFF)
                plsc.addupdate_scatter(hist_v, [digit * L + lane], ones_i, mask=msk)

            d, cgt = select_digit(kk)
            prefix = prefix | jnp.left_shift(d, shift_v)
            kk = kk - cgt
            return prefix, kk

        prefix, kk = lax.fori_loop(1, 4, pass_body, (prefix, kk))

        @plsc.parallel_loop(0, NV, unroll=8)
        def _(i):
            key = key_v[pl.ds(i * L, L)]
            row_v[0, pl.ds(i * L, L)] = jnp.where(key >= prefix, ones_f, zeros_f)

        pltpu.sync_copy(row_v, out_hbm.at[pl.ds(base + r, 1)])
        return 0

    lax.fori_loop(0, ROWS_PER_W, row_body, 0)


_mesh = plsc.VectorSubcoreMesh(core_axis_name="c", subcore_axis_name="s")


@functools.partial(
    pl.kernel,
    out_type=jax.ShapeDtypeStruct((B, N), jnp.float32),
    mesh=_mesh,
    scratch_types=[
        pltpu.VMEM((1, N), jnp.float32),
        pltpu.VMEM((N,), jnp.int32),
        pltpu.VMEM((256 * L,), jnp.int32),
        pltpu.VMEM((256,), jnp.int32),
    ],
    compiler_params=pltpu.CompilerParams(needs_layout_passes=False),
)
def _topk_mask(x_hbm, out_hbm, row_v, key_v, hist_v, tr_v):
    _tec_body(x_hbm, out_hbm, row_v, key_v, hist_v, tr_v)


def kernel(inputs):
    return _topk_mask(inputs)


# speedup vs baseline: 9.1936x; 1.0364x over previous
import functools

import jax
import jax.numpy as jnp
from jax import lax
from jax.experimental import pallas as pl
from jax.experimental.pallas import tpu as pltpu
from jax.experimental.pallas import tpu_sc as plsc

B = 64
N = 8192
K = 256
L = 16
NV = N // L
NC = 2
NW = 32
ROWS_PER_W = B // NW
HW = 256 * L


def _tec_body(x_hbm, out_hbm, row0_v, row1_v, key0_v, key1_v, hist_v,
              tr0_v, tr1_v, sem0, sem1):
    wid = lax.axis_index("s") * NC + lax.axis_index("c")
    base = wid * ROWS_PER_W

    in0 = pltpu.async_copy(x_hbm.at[pl.ds(base, 1)], row0_v, sem0)
    in1 = pltpu.async_copy(x_hbm.at[pl.ds(base + 1, 1)], row1_v, sem1)

    lane = lax.iota(jnp.int32, L)
    ones_i = jnp.full((L,), 1, jnp.int32)
    zeros_i = jnp.zeros((L,), jnp.int32)
    ones_f = jnp.full((L,), 1.0, jnp.float32)
    zeros_f = jnp.zeros((L,), jnp.float32)

    def level_select(tot16, kk):
        cs = plsc.cumsum(lax.rev(tot16, (0,)))
        m = cs >= kk
        npc = plsc.all_reduce_population_count(m)
        slot_v = npc - 1
        i0 = 16 - npc
        csm1 = cs.at[jnp.maximum(i0 - 1, 0)].get(mode="promise_in_bounds")
        cgt = jnp.where(i0 == 0, zeros_i, csm1)
        return slot_v, cgt

    def totals(tr_v, offset_fn, vecs_per_group):
        def tree_sum(vs):
            while len(vs) > 1:
                vs = [vs[i] + vs[i + 1] for i in range(0, len(vs) - 1, 2)] + (
                    [vs[-1]] if len(vs) % 2 else [])
            return vs[0]

        for g in range(16):
            acc = tree_sum([hist_v[pl.ds(offset_fn(g, j), L)]
                            for j in range(vecs_per_group)])
            plsc.store_scatter(tr_v, [lane * 16 + g], acc)
        return tree_sum([tr_v[pl.ds(l * L, L)] for l in range(16)])

    def select_digit(hbase, tr_v, kk):
        tot_chunk = totals(tr_v, lambda g, j: hbase + (g * 16 + j) * L, 16)
        cslot_v, cgt_c = level_select(tot_chunk, kk)
        cslot_s = jnp.max(cslot_v)
        ibase = hbase + cslot_s * 256
        tot_in = totals(tr_v, lambda g, j: ibase + g * L, 1)
        kk2 = kk - cgt_c
        islot_v, cgt_i = level_select(tot_in, kk2)
        return cslot_v * 16 + islot_v, cgt_c + cgt_i

    def zero_hists():
        @plsc.parallel_loop(0, 2 * 256, unroll=8)
        def _(i):
            hist_v[pl.ds(i * L, L)] = zeros_i

    def scan_pass0(row_v, key_v, hbase):
        @plsc.parallel_loop(0, NV, unroll=8)
        def _(i):
            x = row_v[0, pl.ds(i * L, L)]
            b = lax.bitcast_convert_type(x, jnp.int32)
            m = lax.shift_right_arithmetic(b, 31) & jnp.int32(0x7FFFFFFF)
            key = b ^ m
            key_v[pl.ds(i * L, L)] = key
            digit = lax.shift_right_arithmetic(key, 24) & jnp.int32(0xFF)
            digit = digit ^ jnp.int32(0x80)
            plsc.addupdate_scatter(hist_v, [hbase + digit * L + lane], ones_i)

    def scan_ref(key_v, hbase, sh8_v, shift_v, pref_hi):
        @plsc.parallel_loop(0, NV, unroll=8)
        def _(i):
            key = key_v[pl.ds(i * L, L)]
            msk = lax.shift_right_arithmetic(key, sh8_v) == pref_hi
            digit = lax.shift_right_arithmetic(key, shift_v) & jnp.int32(0xFF)
            plsc.addupdate_scatter(
                hist_v, [hbase + digit * L + lane], ones_i, mask=msk)

    zero_hists()
    kk0 = jnp.full((L,), K, jnp.int32)
    kk1 = kk0

    in0.wait()
    scan_pass0(row0_v, key0_v, 0)
    in1.wait()
    scan_pass0(row1_v, key1_v, HW)

    d0, cgt0 = select_digit(0, tr0_v, kk0)
    d1, cgt1 = select_digit(HW, tr1_v, kk1)
    pref0 = jnp.left_shift(d0 ^ jnp.int32(0x80), 24)
    pref1 = jnp.left_shift(d1 ^ jnp.int32(0x80), 24)
    kk0 = kk0 - cgt0
    kk1 = kk1 - cgt1

    def pass_body(p, carry):
        pref0, kk0, pref1, kk1 = carry
        shift = 24 - 8 * p
        shift_v = jnp.full((L,), 1, jnp.int32) * shift
        sh8_v = shift_v + 8
        zero_hists()
        scan_ref(key0_v, 0, sh8_v, shift_v,
                 lax.shift_right_arithmetic(pref0, sh8_v))
        scan_ref(key1_v, HW, sh8_v, shift_v,
                 lax.shift_right_arithmetic(pref1, sh8_v))
        d0, cgt0 = select_digit(0, tr0_v, kk0)
        d1, cgt1 = select_digit(HW, tr1_v, kk1)
        pref0 = pref0 | jnp.left_shift(d0, shift_v)
        pref1 = pref1 | jnp.left_shift(d1, shift_v)
        return pref0, kk0 - cgt0, pref1, kk1 - cgt1

    pref0, kk0, pref1, kk1 = lax.fori_loop(
        1, 4, pass_body, (pref0, kk0, pref1, kk1))

    @plsc.parallel_loop(0, NV, unroll=8)
    def _(i):
        key = key0_v[pl.ds(i * L, L)]
        row0_v[0, pl.ds(i * L, L)] = jnp.where(key >= pref0, ones_f, zeros_f)

    out0 = pltpu.async_copy(row0_v, out_hbm.at[pl.ds(base, 1)], sem0)

    @plsc.parallel_loop(0, NV, unroll=8)
    def _(i):
        key = key1_v[pl.ds(i * L, L)]
        row1_v[0, pl.ds(i * L, L)] = jnp.where(key >= pref1, ones_f, zeros_f)

    out1 = pltpu.async_copy(row1_v, out_hbm.at[pl.ds(base + 1, 1)], sem1)
    out0.wait()
    out1.wait()


_mesh = plsc.VectorSubcoreMesh(core_axis_name="c", subcore_axis_name="s")


@functools.partial(
    pl.kernel,
    out_type=jax.ShapeDtypeStruct((B, N), jnp.float32),
    mesh=_mesh,
    scratch_types=[
        pltpu.VMEM((1, N), jnp.float32),
        pltpu.VMEM((1, N), jnp.float32),
        pltpu.VMEM((N,), jnp.int32),
        pltpu.VMEM((N,), jnp.int32),
        pltpu.VMEM((2 * HW,), jnp.int32),
        pltpu.VMEM((256,), jnp.int32),
        pltpu.VMEM((256,), jnp.int32),
        pltpu.SemaphoreType.DMA,
        pltpu.SemaphoreType.DMA,
    ],
    compiler_params=pltpu.CompilerParams(needs_layout_passes=False),
)
def _topk_mask(x_hbm, out_hbm, row0_v, row1_v, key0_v, key1_v, hist_v,
               tr0_v, tr1_v, sem0, sem1):
    _tec_body(x_hbm, out_hbm, row0_v, row1_v, key0_v, key1_v, hist_v,
              tr0_v, tr1_v, sem0, sem1)


def kernel(inputs):
    return _topk_mask(inputs)


# speedup vs baseline: 9.8475x; 1.0711x over previous
import functools

import jax
import jax.numpy as jnp
from jax import lax
from jax.experimental import pallas as pl
from jax.experimental.pallas import tpu as pltpu
from jax.experimental.pallas import tpu_sc as plsc

B = 64
N = 8192
K = 256
L = 16
NV = N // L
NC = 2
NW = 32
ROWS_PER_W = B // NW
HW = 256 * L


def _tec_body(x_hbm, out_hbm, row0_v, row1_v, key0_v, key1_v, hist_v,
              tr0_v, tr1_v, sem0, sem1):
    wid = lax.axis_index("s") * NC + lax.axis_index("c")
    base = wid * ROWS_PER_W

    in0 = pltpu.async_copy(x_hbm.at[pl.ds(base, 1)], row0_v, sem0)
    in1 = pltpu.async_copy(x_hbm.at[pl.ds(base + 1, 1)], row1_v, sem1)

    lane = lax.iota(jnp.int32, L)
    ones_i = jnp.full((L,), 1, jnp.int32)
    zeros_i = jnp.zeros((L,), jnp.int32)
    ones_f = jnp.full((L,), 1.0, jnp.float32)
    zeros_f = jnp.zeros((L,), jnp.float32)

    def level_select(tot16, kk):
        cs = plsc.cumsum(lax.rev(tot16, (0,)))
        m = cs >= kk
        npc = plsc.all_reduce_population_count(m)
        slot_v = npc - 1
        i0 = 16 - npc
        csm1 = cs.at[jnp.maximum(i0 - 1, 0)].get(mode="promise_in_bounds")
        cgt = jnp.where(i0 == 0, zeros_i, csm1)
        return slot_v, cgt

    def totals(tr_v, offset_fn, vecs_per_group):
        def tree_sum(vs):
            while len(vs) > 1:
                vs = [vs[i] + vs[i + 1] for i in range(0, len(vs) - 1, 2)] + (
                    [vs[-1]] if len(vs) % 2 else [])
            return vs[0]

        for g in range(16):
            acc = tree_sum([hist_v[pl.ds(offset_fn(g, j), L)]
                            for j in range(vecs_per_group)])
            plsc.store_scatter(tr_v, [lane * 16 + g], acc)
        return tree_sum([tr_v[pl.ds(l * L, L)] for l in range(16)])

    def select_digit(hbase, tr_v, kk):
        tot_chunk = totals(tr_v, lambda g, j: hbase + (g * 16 + j) * L, 16)
        cslot_v, cgt_c = level_select(tot_chunk, kk)
        cslot_s = jnp.max(cslot_v)
        ibase = hbase + cslot_s * 256
        tot_in = totals(tr_v, lambda g, j: ibase + g * L, 1)
        kk2 = kk - cgt_c
        islot_v, cgt_i = level_select(tot_in, kk2)
        return cslot_v * 16 + islot_v, cgt_c + cgt_i

    def zero_hists():
        @plsc.parallel_loop(0, 2 * 256, unroll=8)
        def _(i):
            hist_v[pl.ds(i * L, L)] = zeros_i

    def scan_pass0(row_v, key_v, hbase):
        @plsc.parallel_loop(0, NV, unroll=8)
        def _(i):
            x = row_v[0, pl.ds(i * L, L)]
            b = lax.bitcast_convert_type(x, jnp.int32)
            m = lax.shift_right_arithmetic(b, 31) & jnp.int32(0x7FFFFFFF)
            key = b ^ m
            key_v[pl.ds(i * L, L)] = key
            digit = lax.shift_right_arithmetic(key, 24) & jnp.int32(0xFF)
            digit = digit ^ jnp.int32(0x80)
            plsc.addupdate_scatter(hist_v, [hbase + digit * L + lane], ones_i)

    def scan_ref(key_v, hbase, sh8_v, shift_v, pref_hi):
        @plsc.parallel_loop(0, NV, unroll=8)
        def _(i):
            key = key_v[pl.ds(i * L, L)]
            msk = lax.shift_right_arithmetic(key, sh8_v) == pref_hi
            digit = lax.shift_right_arithmetic(key, shift_v) & jnp.int32(0xFF)
            plsc.addupdate_scatter(
                hist_v, [hbase + digit * L + lane], ones_i, mask=msk)

    kk0 = jnp.full((L,), K, jnp.int32)
    kk1 = kk0
    pref0 = zeros_i
    pref1 = zeros_i

    def pass_body(p, carry):
        pref0, kk0, pref1, kk1 = carry
        shift = 24 - 8 * p
        shift_v = jnp.full((L,), 1, jnp.int32) * shift
        sh8_v = shift_v + 8
        zero_hists()

        def first(_):
            in0.wait()
            scan_pass0(row0_v, key0_v, 0)
            in1.wait()
            scan_pass0(row1_v, key1_v, HW)
            return 0

        def later(_):
            scan_ref(key0_v, 0, sh8_v, shift_v,
                     lax.shift_right_arithmetic(pref0, sh8_v))
            scan_ref(key1_v, HW, sh8_v, shift_v,
                     lax.shift_right_arithmetic(pref1, sh8_v))
            return 0

        lax.cond(p == 0, first, later, 0)
        d0, cgt0 = select_digit(0, tr0_v, kk0)
        d1, cgt1 = select_digit(HW, tr1_v, kk1)
        flip = jnp.where(p == 0, jnp.int32(0x80), jnp.int32(0))
        pref0 = pref0 | jnp.left_shift(d0 ^ flip, shift_v)
        pref1 = pref1 | jnp.left_shift(d1 ^ flip, shift_v)
        return pref0, kk0 - cgt0, pref1, kk1 - cgt1

    pref0, kk0, pref1, kk1 = lax.fori_loop(
        0, 4, pass_body, (pref0, kk0, pref1, kk1))

    @plsc.parallel_loop(0, NV, unroll=8)
    def _(i):
        key = key0_v[pl.ds(i * L, L)]
        row0_v[0, pl.ds(i * L, L)] = jnp.where(key >= pref0, ones_f, zeros_f)

    out0 = pltpu.async_copy(row0_v, out_hbm.at[pl.ds(base, 1)], sem0)

    @plsc.parallel_loop(0, NV, unroll=8)
    def _(i):
        key = key1_v[pl.ds(i * L, L)]
        row1_v[0, pl.ds(i * L, L)] = jnp.where(key >= pref1, ones_f, zeros_f)

    out1 = pltpu.async_copy(row1_v, out_hbm.at[pl.ds(base + 1, 1)], sem1)
    out0.wait()
    out1.wait()


_mesh = plsc.VectorSubcoreMesh(core_axis_name="c", subcore_axis_name="s")


@functools.partial(
    pl.kernel,
    out_type=jax.ShapeDtypeStruct((B, N), jnp.float32),
    mesh=_mesh,
    scratch_types=[
        pltpu.VMEM((1, N), jnp.float32),
        pltpu.VMEM((1, N), jnp.float32),
        pltpu.VMEM((N,), jnp.int32),
        pltpu.VMEM((N,), jnp.int32),
        pltpu.VMEM((2 * HW,), jnp.int32),
        pltpu.VMEM((256,), jnp.int32),
        pltpu.VMEM((256,), jnp.int32),
        pltpu.SemaphoreType.DMA,
        pltpu.SemaphoreType.DMA,
    ],
    compiler_params=pltpu.CompilerParams(
        needs_layout_passes=False, skip_device_barrier=True),
)
def _topk_mask(x_hbm, out_hbm, row0_v, row1_v, key0_v, key1_v, hist_v,
               tr0_v, tr1_v, sem0, sem1):
    _tec_body(x_hbm, out_hbm, row0_v, row1_v, key0_v, key1_v, hist_v,
              tr0_v, tr1_v, sem0, sem1)


def kernel(inputs):
    return _topk_mask(inputs)
